# Initial kernel scaffold; baseline (speedup 1.0000x reference)
import jax
import jax.numpy as jnp
from jax.experimental import pallas as pl


def kernel(inp, table):
    raise NotImplementedError("write your pallas kernel here")


# speedup vs baseline: 9.1861x; 9.1861x over previous
import functools

import jax
import jax.numpy as jnp
from jax import lax
from jax.experimental import pallas as pl
from jax.experimental.pallas import tpu as pltpu
from jax.experimental.pallas import tpu_sc as plsc

_NC = 2
_NS = 16
_NW = _NC * _NS
_C = 128
_NBUF = 4
_AHEAD = 2


@functools.partial(jax.jit, static_argnames=())
def _sc_gather(table, idx):
    nw, nchunks, c = idx.shape
    v, d = table.shape
    b_per_w = nchunks * c
    b_total = nw * b_per_w
    mesh = plsc.VectorSubcoreMesh(core_axis_name="c", subcore_axis_name="s")

    @functools.partial(
        pl.kernel,
        out_type=jax.ShapeDtypeStruct((b_total, d), table.dtype),
        mesh=mesh,
        scratch_types=[
            pltpu.VMEM((nchunks, c), jnp.int32),
            pltpu.VMEM((_NBUF, c, d), table.dtype),
            pltpu.SemaphoreType.DMA((_NBUF,)),
            pltpu.SemaphoreType.DMA((_NBUF,)),
        ],
    )
    def body(table_hbm, idx_hbm, out_hbm, idx_v, rows_v, gsem, ssem):
        wid = lax.axis_index("s") * _NC + lax.axis_index("c")
        base = wid * b_per_w

        pltpu.sync_copy(idx_hbm.at[wid], idx_v)

        for b in range(_AHEAD):
            pltpu.async_copy(
                table_hbm.at[idx_v.at[b]], rows_v.at[b], gsem.at[b])

        @pl.loop(0, nchunks, step=_NBUF)
        def _(g0):
            for b in range(_NBUF):
                g = g0 + b
                pltpu.make_async_copy(
                    table_hbm.at[pl.ds(0, c)], rows_v.at[b], gsem.at[b]
                ).wait()
                pltpu.async_copy(
                    rows_v.at[b],
                    out_hbm.at[pl.ds(base + g * c, c)],
                    ssem.at[b],
                )
                h = g + _AHEAD
                hb = (b + _AHEAD) % _NBUF

                @pl.when(h < nchunks)
                def _():
                    @pl.when(h >= _NBUF)
                    def _():
                        pltpu.make_async_copy(
                            rows_v.at[hb],
                            out_hbm.at[pl.ds(base, c)],
                            ssem.at[hb],
                        ).wait()

                    pltpu.async_copy(
                        table_hbm.at[idx_v.at[h]], rows_v.at[hb], gsem.at[hb])

        for b in range(_NBUF):
            pltpu.make_async_copy(
                rows_v.at[b], out_hbm.at[pl.ds(base, c)], ssem.at[b]
            ).wait()

    return body(table, idx)


def kernel(inp, table):
    b0, s = inp.shape
    d = table.shape[1]
    idx = inp.astype(jnp.int32).reshape(_NW, -1, _C)
    out = _sc_gather(table, idx)
    return out.reshape(b0, s, d)


# speedup vs baseline: 9.1931x; 1.0008x over previous
import functools

import jax
import jax.numpy as jnp
from jax import lax
from jax.experimental import pallas as pl
from jax.experimental.pallas import tpu as pltpu
from jax.experimental.pallas import tpu_sc as plsc

_NC = 2
_NS = 16
_NW = _NC * _NS
_C = 128
_NBUF = 5
_AHEAD = 3


@functools.partial(jax.jit, static_argnames=())
def _sc_gather(table, idx):
    nw, nchunks, c = idx.shape
    v, d = table.shape
    b_per_w = nchunks * c
    b_total = nw * b_per_w
    mesh = plsc.VectorSubcoreMesh(core_axis_name="c", subcore_axis_name="s")

    @functools.partial(
        pl.kernel,
        out_type=jax.ShapeDtypeStruct((b_total, d), table.dtype),
        mesh=mesh,
        scratch_types=[
            pltpu.VMEM((nchunks, c), jnp.int32),
            pltpu.VMEM((_NBUF, c, d), table.dtype),
            pltpu.SemaphoreType.DMA((_NBUF,)),
            pltpu.SemaphoreType.DMA((_NBUF,)),
        ],
    )
    def body(table_hbm, idx_hbm, out_hbm, idx_v, rows_v, gsem, ssem):
        wid = lax.axis_index("s") * _NC + lax.axis_index("c")
        base = wid * b_per_w

        pltpu.sync_copy(idx_hbm.at[wid], idx_v)

        for b in range(_AHEAD):
            pltpu.async_copy(
                table_hbm.at[idx_v.at[b]], rows_v.at[b], gsem.at[b])

        @pl.loop(0, nchunks, step=_NBUF)
        def _(g0):
            for b in range(_NBUF):
                g = g0 + b
                pltpu.make_async_copy(
                    table_hbm.at[pl.ds(0, c)], rows_v.at[b], gsem.at[b]
                ).wait()
                pltpu.async_copy(
                    rows_v.at[b],
                    out_hbm.at[pl.ds(base + g * c, c)],
                    ssem.at[b],
                )
                h = g + _AHEAD
                hb = (b + _AHEAD) % _NBUF

                @pl.when(h < nchunks)
                def _():
                    @pl.when(h >= _NBUF)
                    def _():
                        pltpu.make_async_copy(
                            rows_v.at[hb],
                            out_hbm.at[pl.ds(base, c)],
                            ssem.at[hb],
                        ).wait()

                    pltpu.async_copy(
                        table_hbm.at[idx_v.at[h]], rows_v.at[hb], gsem.at[hb])

        for b in range(_NBUF):
            pltpu.make_async_copy(
                rows_v.at[b], out_hbm.at[pl.ds(base, c)], ssem.at[b]
            ).wait()

    return body(table, idx)


def kernel(inp, table):
    b0, s = inp.shape
    d = table.shape[1]
    idx = inp.astype(jnp.int32).reshape(_NW, -1, _C)
    out = _sc_gather(table, idx)
    return out.reshape(b0, s, d)
